# baseline (speedup 1.0000x reference)
import jax
import jax.numpy as jnp
from jax import lax
from jax.experimental import pallas as pl
from jax.experimental.pallas import tpu as pltpu
from jax.experimental.pallas import tpu_sc as plsc

B = 16384
N_NUM = 13
N_CAT = 26
VOCAB = 100000
EMB_DIM = 32
EPS = 1e-08
N_COLS = N_NUM + N_CAT
OUT_D = N_NUM + N_CAT * EMB_DIM

NC = 2
NS = 16
NW = NC * NS
B_PER_W = B // NW
R = 16
N_CHUNKS = B_PER_W // R
LANES = 16
HALF = EMB_DIM // LANES
SUPER_D = 128
SUPER_ROWS = N_CAT * VOCAB * EMB_DIM // SUPER_D


def _body(x_hbm, tab_hbm, mean_hbm, std_hbm, out_hbm,
          xbuf, idx2d, sub2d, catbuf, obuf, mean_v, std_v, gsem):
  wid = lax.axis_index("s") * NC + lax.axis_index("c")

  pltpu.sync_copy(mean_hbm, mean_v)
  pltpu.sync_copy(std_hbm, std_v)

  iota = lax.iota(jnp.int32, LANES)

  def chunk_body(ch, carry):
    base = wid * B_PER_W + ch * R

    pltpu.sync_copy(x_hbm.at[pl.ds(base, R)], xbuf)

    for f in range(N_CAT):
      col = jnp.full((LANES,), N_NUM + f, jnp.int32)
      ids_f = plsc.load_gather(xbuf, [iota, col])
      gid = ids_f.astype(jnp.int32) + (f * VOCAB)
      idx2d[f] = lax.shift_right_logical(gid, 2)
      sub2d[f] = lax.bitwise_and(gid, 3)

    copies = []
    for f in range(N_CAT):
      copies.append(pltpu.async_copy(
          tab_hbm.at[idx2d.at[f]], catbuf.at[f], gsem))

    for c in range(N_NUM):
      colv = jnp.full((LANES,), c, jnp.int32)
      m = plsc.load_gather(mean_v, [colv])
      s = plsc.load_gather(std_v, [colv]) + EPS
      v = plsc.load_gather(xbuf, [iota, colv])
      plsc.store_scatter(obuf, [iota, colv], (v - m) / s)

    for d in copies:
      d.wait()

    def row_body(r, carry):
      rv = jnp.full((LANES,), r, jnp.int32)
      for f in range(N_CAT):
        fv = jnp.full((LANES,), f, jnp.int32)
        sub = plsc.load_gather(sub2d, [fv, rv])
        src0 = sub * EMB_DIM + iota
        for h in range(HALF):
          src_c = src0 + (h * LANES)
          dst_c = iota + (N_NUM + f * EMB_DIM + h * LANES)
          v = plsc.load_gather(catbuf, [fv, rv, src_c])
          plsc.store_scatter(obuf, [rv, dst_c], v)
      return carry

    lax.fori_loop(0, R, row_body, 0)

    pltpu.sync_copy(obuf, out_hbm.at[pl.ds(base, R)])
    return carry

  lax.fori_loop(0, N_CHUNKS, chunk_body, 0)


@jax.jit
def _run(x, tab_flat, mean16, std16):
  mesh = plsc.VectorSubcoreMesh(core_axis_name="c", subcore_axis_name="s",
                                num_cores=NC, num_subcores=NS)
  return pl.kernel(
      _body,
      out_type=jax.ShapeDtypeStruct((B, OUT_D), jnp.float32),
      mesh=mesh,
      compiler_params=pltpu.CompilerParams(needs_layout_passes=False),
      scratch_types=[
          pltpu.VMEM((R, N_COLS), jnp.float32),
          pltpu.VMEM((N_CAT, R), jnp.int32),
          pltpu.VMEM((N_CAT, R), jnp.int32),
          pltpu.VMEM((N_CAT, R, SUPER_D), jnp.float32),
          pltpu.VMEM((R, OUT_D), jnp.float32),
          pltpu.VMEM((LANES,), jnp.float32),
          pltpu.VMEM((LANES,), jnp.float32),
          pltpu.SemaphoreType.DMA,
      ],
  )(x, tab_flat, mean16, std16)


def kernel(x, tables, mean, std):
  tab_flat = tables.reshape(SUPER_ROWS, SUPER_D)
  mean16 = jnp.zeros((LANES,), jnp.float32).at[:N_NUM].set(mean)
  std16 = jnp.ones((LANES,), jnp.float32).at[:N_NUM].set(std)
  return _run(x, tab_flat, mean16, std16)
